# baseline (speedup 1.0000x reference)
import functools

import jax
import jax.numpy as jnp
from jax import lax
from jax.experimental import pallas as pl
from jax.experimental.pallas import tpu as pltpu
from jax.experimental.pallas import tpu_sc as plsc

NUM_CORES = 2
NUM_SUBCORES = 16
NUM_WORKERS = NUM_CORES * NUM_SUBCORES
NBUF = 5
GLOOK = 4
WLAG = NBUF - GLOOK


@jax.jit
def _embedding_lookup(weight, token_ids):
    tok_t = token_ids.astype(jnp.int32).T
    seq, batch = tok_t.shape
    _, dim = weight.shape
    n_chunks = seq
    n_groups = n_chunks // NBUF
    blk = batch // NUM_WORKERS
    mesh = plsc.VectorSubcoreMesh(core_axis_name="c", subcore_axis_name="s")

    @functools.partial(
        pl.kernel,
        mesh=mesh,
        out_type=jax.ShapeDtypeStruct((seq, batch, dim), jnp.float32),
        scratch_types=[pltpu.VMEM((seq, blk), jnp.int32)]
        + [pltpu.VMEM((blk, dim), jnp.float32)] * NBUF
        + [pltpu.SemaphoreType.DMA] * (2 * NBUF),
        compiler_params=pltpu.CompilerParams(
            use_tc_tiling_on_sc=True, skip_device_barrier=True
        ),
    )
    def k(table_hbm, tok_hbm, out_hbm, idx_v, *rows_and_sems):
        rows = rows_and_sems[:NBUF]
        gsem = rows_and_sems[NBUF : 2 * NBUF]
        wsem = rows_and_sems[2 * NBUF :]
        wid = lax.axis_index("s") * NUM_CORES + lax.axis_index("c")
        b0 = wid * blk
        pltpu.sync_copy(tok_hbm.at[:, pl.ds(b0, blk)], idx_v)

        def gather(chunk, b):
            return pltpu.make_async_copy(
                table_hbm.at[idx_v.at[chunk]], rows[b], gsem[b]
            )

        def write(chunk, b):
            return pltpu.make_async_copy(
                rows[b], out_hbm.at[chunk, pl.ds(b0, blk)], wsem[b]
            )

        def step(c, b, wait_w, start_g):
            gather(c, b).wait()
            write(c, b).start()
            if wait_w:
                write(c - WLAG, (b - WLAG) % NBUF).wait()
            if start_g:
                gather(c + GLOOK, (b + GLOOK) % NBUF).start()

        for c in range(GLOOK):
            gather(c, c).start()

        for b in range(NBUF):
            step(b, b, b >= WLAG, True)

        def body(g, carry):
            c0 = g * NBUF
            for b in range(NBUF):
                step(c0 + b, b, True, True)
            return carry

        lax.fori_loop(1, n_groups - 1, body, 0)

        c0 = (n_groups - 1) * NBUF
        for b in range(NBUF):
            step(c0 + b, b, True, b + GLOOK < NBUF)

        for c in range(n_chunks - WLAG, n_chunks):
            write(c, c % NBUF).wait()

    out_t = k(weight, tok_t)
    return jnp.transpose(out_t, (1, 0, 2))


def kernel(token_ids, weight):
    return _embedding_lookup(weight, token_ids)


# speedup vs baseline: 1.0035x; 1.0035x over previous
import functools

import jax
import jax.numpy as jnp
from jax import lax
from jax.experimental import pallas as pl
from jax.experimental.pallas import tpu as pltpu
from jax.experimental.pallas import tpu_sc as plsc

NUM_CORES = 2
NUM_SUBCORES = 16
NUM_WORKERS = NUM_CORES * NUM_SUBCORES
CH = 64
NBUF = 10
GLOOK = 7
WLAG = NBUF - GLOOK


@jax.jit
def _embedding_lookup(weight, token_ids):
    tok_t = token_ids.astype(jnp.int32).T
    seq, batch = tok_t.shape
    _, dim = weight.shape
    blk = batch // NUM_WORKERS
    per_row = blk // CH
    n_chunks = seq * per_row
    n_groups = n_chunks // NBUF
    mesh = plsc.VectorSubcoreMesh(core_axis_name="c", subcore_axis_name="s")

    @functools.partial(
        pl.kernel,
        mesh=mesh,
        out_type=jax.ShapeDtypeStruct((seq, batch, dim), jnp.float32),
        scratch_types=[pltpu.VMEM((seq, blk), jnp.int32)]
        + [pltpu.VMEM((CH, dim), jnp.float32)] * NBUF
        + [pltpu.SemaphoreType.DMA] * (2 * NBUF),
        compiler_params=pltpu.CompilerParams(use_tc_tiling_on_sc=True),
    )
    def k(table_hbm, tok_hbm, out_hbm, idx_v, *rows_and_sems):
        rows = rows_and_sems[:NBUF]
        gsem = rows_and_sems[NBUF : 2 * NBUF]
        wsem = rows_and_sems[2 * NBUF :]
        wid = lax.axis_index("s") * NUM_CORES + lax.axis_index("c")
        b0 = wid * blk
        pltpu.sync_copy(tok_hbm.at[:, pl.ds(b0, blk)], idx_v)

        def gather(chunk, b):
            t = chunk // per_row
            off = (chunk % per_row) * CH
            return pltpu.make_async_copy(
                table_hbm.at[idx_v.at[t, pl.ds(off, CH)]], rows[b], gsem[b]
            )

        def write(chunk, b):
            t = chunk // per_row
            off = (chunk % per_row) * CH
            return pltpu.make_async_copy(
                rows[b], out_hbm.at[t, pl.ds(b0 + off, CH)], wsem[b]
            )

        def step(c, b, wait_w, start_g):
            gather(c, b).wait()
            write(c, b).start()
            if wait_w:
                write(c - WLAG, (b - WLAG) % NBUF).wait()
            if start_g:
                gather(c + GLOOK, (b + GLOOK) % NBUF).start()

        for c in range(GLOOK):
            gather(c, c).start()

        for b in range(NBUF):
            step(b, b, b >= WLAG, True)

        def body(g, carry):
            c0 = g * NBUF
            for b in range(NBUF):
                step(c0 + b, b, True, True)
            return carry

        lax.fori_loop(1, n_groups - 1, body, 0)

        c0 = (n_groups - 1) * NBUF
        for b in range(NBUF):
            step(c0 + b, b, True, b + GLOOK < NBUF)

        for c in range(n_chunks - WLAG, n_chunks):
            write(c, c % NBUF).wait()

    out_t = k(weight, tok_t)
    return jnp.transpose(out_t, (1, 0, 2))


def kernel(token_ids, weight):
    return _embedding_lookup(weight, token_ids)
